# baseline (speedup 1.0000x reference)
import functools

import ml_dtypes
import numpy as np
import jax
import jax.numpy as jnp
from jax import lax
from jax.experimental import pallas as pl
from jax.experimental.pallas import tpu as pltpu
from jax.experimental.pallas import tpu_sc as plsc

_K_RATIO = 0.5
_KSIZE = 5
_SIGMA = 1.0


def _gauss_2d_bf16():
    c = _KSIZE // 2
    x, y = np.mgrid[0 - c:_KSIZE - c, 0 - c:_KSIZE - c]
    gk = 1.0 / (2.0 * np.pi * _SIGMA) * np.exp(
        -(np.square(x) + np.square(y)) / (2.0 * np.square(_SIGMA)))
    gk32 = gk.astype(np.float32)
    return gk32.astype(ml_dtypes.bfloat16).astype(np.float32)


def _tc_body(bands_ref, conf_ref, wm_ref, bits_ref, thr_ref, pad_ref, cm_ref):
    A = conf_ref.shape[2]
    Lm1, H, W = cm_ref.shape
    K = int(H * W * _K_RATIO)
    P = _KSIZE // 2

    m = conf_ref[0, 1:, 0]
    for a in range(1, A):
        m = jnp.maximum(m, conf_ref[0, 1:, a])
    s = jax.nn.sigmoid(m) * wm_ref[0, 1:, 0]

    pad_ref[...] = jnp.zeros_like(pad_ref)
    pad_ref[:, P:P + H, P:P + W] = s.astype(jnp.bfloat16)
    for l in range(Lm1):
        acc = jnp.zeros((H, W), jnp.float32)
        for dy in range(_KSIZE):
            acc += lax.dot_general(
                pad_ref[l, dy:dy + H, :], bands_ref[dy],
                (((1,), (0,)), ((), ())),
                preferred_element_type=jnp.float32)
        cm_ref[l] = acc

    bits = lax.bitcast_convert_type(cm_ref[...], jnp.int32)
    bits_ref[0] = bits

    def step(_, lohi):
        lo, hi = lohi
        mid = lo + (hi - lo + 1) // 2
        cnt = jnp.sum((bits >= mid).astype(jnp.int32), axis=(1, 2),
                      keepdims=True)
        big = cnt >= K
        return jnp.where(big, mid, lo), jnp.where(big, hi, mid - 1)

    lo0 = jnp.zeros((Lm1, 1, 1), jnp.int32)
    hi0 = jnp.full((Lm1, 1, 1), 0x3F800000, jnp.int32)
    lo, _ = lax.fori_loop(0, 30, step, (lo0, hi0))
    thr_ref[0] = jnp.broadcast_to(lo.reshape(Lm1, 1), (Lm1, thr_ref.shape[2]))


def _sc_mask_kernel(n_maps, n_elems):
    mesh = plsc.VectorSubcoreMesh(core_axis_name="c", subcore_axis_name="s")

    @functools.partial(
        pl.kernel, mesh=mesh,
        out_type=[
            jax.ShapeDtypeStruct((n_maps, n_elems), jnp.float32),
            jax.ShapeDtypeStruct((n_maps, 16), jnp.int32),
        ],
        scratch_types=[
            pltpu.VMEM((1, n_elems), jnp.int32),
            pltpu.VMEM((1, n_elems), jnp.float32),
            pltpu.VMEM((1, 16), jnp.int32),
            pltpu.VMEM((1, 16), jnp.int32),
        ],
    )
    def k(bits_hbm, thr_hbm, mask_hbm, cnt_hbm, buf, outb, thrb, cntb):
        c = lax.axis_index("c")
        s = lax.axis_index("s")
        w = s * 2 + c

        @pl.when(w < n_maps)
        def _():
            pltpu.sync_copy(bits_hbm.at[pl.ds(w, 1)], buf)
            pltpu.sync_copy(thr_hbm.at[pl.ds(w, 1)], thrb)
            t = thrb[0, :]
            one = jnp.ones((16,), jnp.float32)
            zero = jnp.zeros((16,), jnp.float32)

            def body(j, cnt):
                v = buf[0, pl.ds(j * 16, 16)]
                m = v >= t
                outb[0, pl.ds(j * 16, 16)] = jnp.where(m, one, zero)
                return cnt + jnp.where(m, 1, 0)

            cnt = lax.fori_loop(0, n_elems // 16, body,
                                jnp.zeros((16,), jnp.int32))
            cntb[0, :] = cnt
            pltpu.sync_copy(outb, mask_hbm.at[pl.ds(w, 1)])
            pltpu.sync_copy(cntb, cnt_hbm.at[pl.ds(w, 1)])

    return k


def kernel(batch_confidence_maps, B, batch_warp_maks_list, record_len,
           warp_vis_list, warp_conf_list, warp_x_list, gauss_kernel):
    Bs, L, A, H, W = batch_confidence_maps.shape
    P = _KSIZE // 2
    n_maps = Bs * (L - 1)

    gw = _gauss_2d_bf16()
    bands_np = np.zeros((_KSIZE, W + 2 * P, W), np.float32)
    cols = np.arange(W)
    for dy in range(_KSIZE):
        for dx in range(_KSIZE):
            bands_np[dy, cols + dx, cols] = gw[dy, dx]
    bands = jnp.asarray(bands_np, dtype=jnp.bfloat16)

    bits, thr = pl.pallas_call(
        _tc_body,
        grid=(Bs,),
        in_specs=[
            pl.BlockSpec((_KSIZE, W + 2 * P, W), lambda b: (0, 0, 0)),
            pl.BlockSpec((1, L, A, H, W), lambda b: (b, 0, 0, 0, 0)),
            pl.BlockSpec((1, L, 1, H, W), lambda b: (b, 0, 0, 0, 0)),
        ],
        out_specs=[
            pl.BlockSpec((1, L - 1, H, W), lambda b: (b, 0, 0, 0)),
            pl.BlockSpec((1, L - 1, 16), lambda b: (b, 0, 0)),
        ],
        out_shape=[
            jax.ShapeDtypeStruct((Bs, L - 1, H, W), jnp.int32),
            jax.ShapeDtypeStruct((Bs, L - 1, 16), jnp.int32),
        ],
        scratch_shapes=[
            pltpu.VMEM((L - 1, H + 2 * P, W + 2 * P), jnp.bfloat16),
            pltpu.VMEM((L - 1, H, W), jnp.float32),
        ],
        compiler_params=pltpu.CompilerParams(
            dimension_semantics=("arbitrary",)),
    )(bands, batch_confidence_maps, batch_warp_maks_list)

    sc = _sc_mask_kernel(n_maps, H * W)
    masks_ne, counts = sc(bits.reshape(n_maps, H * W),
                          thr.reshape(n_maps, 16))

    masks_ne = masks_ne.reshape(Bs, L - 1, 1, H, W)
    ego = jnp.ones((Bs, 1, 1, H, W), jnp.float32)
    masks = jnp.concatenate([ego, masks_ne], axis=1).reshape(Bs * L, 1, H, W)

    cnt = jnp.sum(counts.astype(jnp.float32), axis=1).reshape(Bs, L - 1)
    rates = jnp.sum(cnt, axis=1) / ((L - 1) * H * W)
    rate = jnp.sum(rates) / Bs
    return masks, rate


# speedup vs baseline: 1.1318x; 1.1318x over previous
import functools

import ml_dtypes
import numpy as np
import jax
import jax.numpy as jnp
from jax import lax
from jax.experimental import pallas as pl
from jax.experimental.pallas import tpu as pltpu
from jax.experimental.pallas import tpu_sc as plsc

_K_RATIO = 0.5
_KSIZE = 5
_SIGMA = 1.0


def _gauss_2d_bf16():
    c = _KSIZE // 2
    x, y = np.mgrid[0 - c:_KSIZE - c, 0 - c:_KSIZE - c]
    gk = 1.0 / (2.0 * np.pi * _SIGMA) * np.exp(
        -(np.square(x) + np.square(y)) / (2.0 * np.square(_SIGMA)))
    gk32 = gk.astype(np.float32)
    return gk32.astype(ml_dtypes.bfloat16).astype(np.float32)


def _tc_body(bands_ref, conf_ref, wm_ref, bits_ref, thr_ref, pad_ref, cm_ref):
    A = conf_ref.shape[2]
    Lm1, H, W = cm_ref.shape
    K = int(H * W * _K_RATIO)
    P = _KSIZE // 2

    m = conf_ref[0, 1:, 0]
    for a in range(1, A):
        m = jnp.maximum(m, conf_ref[0, 1:, a])
    s = jax.nn.sigmoid(m) * wm_ref[0, 1:, 0]

    pad_ref[...] = jnp.zeros_like(pad_ref)
    pad_ref[:, P:P + H, P:P + W] = s.astype(jnp.bfloat16)
    for l in range(Lm1):
        acc = jnp.zeros((H, W), jnp.float32)
        for dy in range(_KSIZE):
            acc += lax.dot_general(
                pad_ref[l, dy:dy + H, :], bands_ref[dy],
                (((1,), (0,)), ((), ())),
                preferred_element_type=jnp.float32)
        cm_ref[l] = acc

    bits = lax.bitcast_convert_type(cm_ref[...], jnp.int32)
    bits_ref[0] = bits

    def step(_, lohi):
        lo, hi = lohi
        mid = lo + (hi - lo + 1) // 2
        cnt = jnp.sum((bits >= mid).astype(jnp.int32), axis=(1, 2),
                      keepdims=True)
        big = cnt >= K
        return jnp.where(big, mid, lo), jnp.where(big, hi, mid - 1)

    lo0 = jnp.zeros((Lm1, 1, 1), jnp.int32)
    hi0 = jnp.full((Lm1, 1, 1), 0x3F800000, jnp.int32)
    lo, _ = lax.fori_loop(0, 30, step, (lo0, hi0))
    thr_ref[0] = jnp.broadcast_to(lo.reshape(Lm1, 1), (Lm1, thr_ref.shape[2]))


def _sc_mask_kernel(n_maps, n_elems):
    mesh = plsc.VectorSubcoreMesh(core_axis_name="c", subcore_axis_name="s")

    half = n_elems // 2
    unroll = 8

    @functools.partial(
        pl.kernel, mesh=mesh,
        out_type=[
            jax.ShapeDtypeStruct((n_maps * 2, half), jnp.float32),
            jax.ShapeDtypeStruct((n_maps * 2, 16), jnp.int32),
        ],
        scratch_types=[
            pltpu.VMEM((1, half), jnp.int32),
            pltpu.VMEM((1, half), jnp.float32),
            pltpu.VMEM((1, 16), jnp.int32),
            pltpu.VMEM((1, 16), jnp.int32),
        ],
    )
    def k(bits_hbm, thr_hbm, mask_hbm, cnt_hbm, buf, outb, thrb, cntb):
        c = lax.axis_index("c")
        s = lax.axis_index("s")
        w = s * 2 + c

        @pl.when(w < n_maps * 2)
        def _():
            pltpu.sync_copy(bits_hbm.at[pl.ds(w, 1)], buf)
            pltpu.sync_copy(thr_hbm.at[pl.ds(w, 1)], thrb)
            t = thrb[0, :]
            one = jnp.ones((16,), jnp.float32)
            zero = jnp.zeros((16,), jnp.float32)

            def body(j, cnt):
                for u in range(unroll):
                    off = (j * unroll + u) * 16
                    v = buf[0, pl.ds(off, 16)]
                    m = v >= t
                    outb[0, pl.ds(off, 16)] = jnp.where(m, one, zero)
                    cnt = cnt + jnp.where(m, 1, 0)
                return cnt

            cnt = lax.fori_loop(0, half // (16 * unroll), body,
                                jnp.zeros((16,), jnp.int32))
            cntb[0, :] = cnt
            pltpu.sync_copy(outb, mask_hbm.at[pl.ds(w, 1)])
            pltpu.sync_copy(cntb, cnt_hbm.at[pl.ds(w, 1)])

    return k


def kernel(batch_confidence_maps, B, batch_warp_maks_list, record_len,
           warp_vis_list, warp_conf_list, warp_x_list, gauss_kernel):
    Bs, L, A, H, W = batch_confidence_maps.shape
    P = _KSIZE // 2
    n_maps = Bs * (L - 1)

    gw = _gauss_2d_bf16()
    bands_np = np.zeros((_KSIZE, W + 2 * P, W), np.float32)
    cols = np.arange(W)
    for dy in range(_KSIZE):
        for dx in range(_KSIZE):
            bands_np[dy, cols + dx, cols] = gw[dy, dx]
    bands = jnp.asarray(bands_np, dtype=jnp.bfloat16)

    bits, thr = pl.pallas_call(
        _tc_body,
        grid=(Bs,),
        in_specs=[
            pl.BlockSpec((_KSIZE, W + 2 * P, W), lambda b: (0, 0, 0)),
            pl.BlockSpec((1, L, A, H, W), lambda b: (b, 0, 0, 0, 0)),
            pl.BlockSpec((1, L, 1, H, W), lambda b: (b, 0, 0, 0, 0)),
        ],
        out_specs=[
            pl.BlockSpec((1, L - 1, H, W), lambda b: (b, 0, 0, 0)),
            pl.BlockSpec((1, L - 1, 16), lambda b: (b, 0, 0)),
        ],
        out_shape=[
            jax.ShapeDtypeStruct((Bs, L - 1, H, W), jnp.int32),
            jax.ShapeDtypeStruct((Bs, L - 1, 16), jnp.int32),
        ],
        scratch_shapes=[
            pltpu.VMEM((L - 1, H + 2 * P, W + 2 * P), jnp.bfloat16),
            pltpu.VMEM((L - 1, H, W), jnp.float32),
        ],
        compiler_params=pltpu.CompilerParams(
            dimension_semantics=("arbitrary",)),
    )(bands, batch_confidence_maps, batch_warp_maks_list)

    sc = _sc_mask_kernel(n_maps, H * W)
    thr2 = jnp.repeat(thr.reshape(n_maps, 16), 2, axis=0)
    masks_ne, counts = sc(bits.reshape(n_maps * 2, (H * W) // 2), thr2)
    masks_ne = masks_ne.reshape(n_maps, H * W)

    masks_ne = masks_ne.reshape(Bs, L - 1, 1, H, W)
    ego = jnp.ones((Bs, 1, 1, H, W), jnp.float32)
    masks = jnp.concatenate([ego, masks_ne], axis=1).reshape(Bs * L, 1, H, W)

    cnt = jnp.sum(counts.astype(jnp.float32).reshape(Bs, L - 1, 32), axis=2)
    rates = jnp.sum(cnt, axis=1) / ((L - 1) * H * W)
    rate = jnp.sum(rates) / Bs
    return masks, rate


# speedup vs baseline: 2.4965x; 2.2058x over previous
import functools

import ml_dtypes
import numpy as np
import jax
import jax.numpy as jnp
from jax.experimental import pallas as pl
from jax.experimental.pallas import tpu as pltpu

_K_RATIO = 0.5
_KSIZE = 5
_SIGMA = 1.0


def _gauss_2d_bf16():
    c = _KSIZE // 2
    x, y = np.mgrid[0 - c:_KSIZE - c, 0 - c:_KSIZE - c]
    gk = 1.0 / (2.0 * np.pi * _SIGMA) * np.exp(
        -(np.square(x) + np.square(y)) / (2.0 * np.square(_SIGMA)))
    gk32 = gk.astype(np.float32)
    return gk32.astype(ml_dtypes.bfloat16).astype(np.float32)


def _batch_body(bands_ref, conf_ref, wm_ref, mask_ref, cnt_ref, pad_ref,
                cm_ref):
    G, L, A = conf_ref.shape[0], conf_ref.shape[1], conf_ref.shape[2]
    H, W = mask_ref.shape[2], mask_ref.shape[3]
    NM = G * (L - 1)
    K = int(H * W * _K_RATIO)
    P = _KSIZE // 2

    m = conf_ref[:, 1:, 0]
    for a in range(1, A):
        m = jnp.maximum(m, conf_ref[:, 1:, a])
    s = jax.nn.sigmoid(m) * wm_ref[:, 1:, 0]

    pad_ref[...] = jnp.zeros_like(pad_ref)
    pad_ref[:, P:P + H, P:P + W] = s.reshape(NM, H, W).astype(jnp.bfloat16)
    for l in range(NM):
        acc = jnp.zeros((H, W), jnp.float32)
        for dy in range(_KSIZE):
            acc += jax.lax.dot_general(
                pad_ref[l, dy:dy + H, :], bands_ref[dy],
                (((1,), (0,)), ((), ())),
                preferred_element_type=jnp.float32)
        cm_ref[l] = acc

    bits = jax.lax.bitcast_convert_type(cm_ref[...], jnp.int32)

    def step(_, lohi):
        lo, hi = lohi
        mid = lo + (hi - lo + 1) // 2
        cnt = jnp.sum((bits >= mid).astype(jnp.int32), axis=(1, 2),
                      keepdims=True)
        big = cnt >= K
        return jnp.where(big, mid, lo), jnp.where(big, hi, mid - 1)

    lo0 = jnp.zeros((NM, 1, 1), jnp.int32)
    hi0 = jnp.full((NM, 1, 1), 0x3F800000, jnp.int32)
    lo, _ = jax.lax.fori_loop(0, 30, step, (lo0, hi0))

    sel = (bits >= lo).astype(jnp.float32)
    cnt = jnp.sum(sel, axis=(1, 2)).reshape(G, L - 1, 1)
    cnt_ref[:, 0] = jnp.zeros((G, cnt_ref.shape[2]), jnp.float32)
    cnt_ref[:, 1:] = jnp.broadcast_to(cnt, (G, L - 1, cnt_ref.shape[2]))
    mask_ref[:, 0] = jnp.ones((G, H, W), jnp.float32)
    mask_ref[:, 1:] = sel.reshape(G, L - 1, H, W)


def kernel(batch_confidence_maps, B, batch_warp_maks_list, record_len,
           warp_vis_list, warp_conf_list, warp_x_list, gauss_kernel):
    Bs, L, A, H, W = batch_confidence_maps.shape
    P = _KSIZE // 2

    gw = _gauss_2d_bf16()
    bands_np = np.zeros((_KSIZE, W + 2 * P, W), np.float32)
    cols = np.arange(W)
    for dy in range(_KSIZE):
        for dx in range(_KSIZE):
            bands_np[dy, cols + dx, cols] = gw[dy, dx]
    bands = jnp.asarray(bands_np, dtype=jnp.bfloat16)

    G = 2 if Bs % 2 == 0 else 1
    masks, counts = pl.pallas_call(
        _batch_body,
        grid=(Bs // G,),
        in_specs=[
            pl.BlockSpec((_KSIZE, W + 2 * P, W), lambda b: (0, 0, 0)),
            pl.BlockSpec((G, L, A, H, W), lambda b: (b, 0, 0, 0, 0)),
            pl.BlockSpec((G, L, 1, H, W), lambda b: (b, 0, 0, 0, 0)),
        ],
        out_specs=[
            pl.BlockSpec((G, L, H, W), lambda b: (b, 0, 0, 0)),
            pl.BlockSpec((G, L, 128), lambda b: (b, 0, 0)),
        ],
        out_shape=[
            jax.ShapeDtypeStruct((Bs, L, H, W), jnp.float32),
            jax.ShapeDtypeStruct((Bs, L, 128), jnp.float32),
        ],
        scratch_shapes=[
            pltpu.VMEM((G * (L - 1), H + 2 * P, W + 2 * P), jnp.bfloat16),
            pltpu.VMEM((G * (L - 1), H, W), jnp.float32),
        ],
        compiler_params=pltpu.CompilerParams(
            dimension_semantics=("arbitrary",)),
    )(bands, batch_confidence_maps, batch_warp_maks_list)

    masks = masks.reshape(Bs * L, 1, H, W)

    counts = counts[:, :, 0]
    rates = jnp.sum(counts[:, 1:], axis=1) / ((L - 1) * H * W)
    rate = jnp.sum(rates) / Bs
    return masks, rate


# speedup vs baseline: 2.8322x; 1.1345x over previous
import functools

import ml_dtypes
import numpy as np
import jax
import jax.numpy as jnp
from jax.experimental import pallas as pl
from jax.experimental.pallas import tpu as pltpu

_K_RATIO = 0.5
_KSIZE = 5
_SIGMA = 1.0


def _gauss_2d_bf16():
    c = _KSIZE // 2
    x, y = np.mgrid[0 - c:_KSIZE - c, 0 - c:_KSIZE - c]
    gk = 1.0 / (2.0 * np.pi * _SIGMA) * np.exp(
        -(np.square(x) + np.square(y)) / (2.0 * np.square(_SIGMA)))
    gk32 = gk.astype(np.float32)
    return gk32.astype(ml_dtypes.bfloat16).astype(np.float32)


def _batch_body(bands_ref, conf_ref, wm_ref, mask_ref, cnt_ref, pad_ref,
                cm_ref):
    G, L, A = conf_ref.shape[0], conf_ref.shape[1], conf_ref.shape[2]
    H, W = mask_ref.shape[2], mask_ref.shape[3]
    NM = G * (L - 1)
    K = int(H * W * _K_RATIO)
    P = _KSIZE // 2

    m = conf_ref[:, 1:, 0]
    for a in range(1, A):
        m = jnp.maximum(m, conf_ref[:, 1:, a])
    s = jax.nn.sigmoid(m) * wm_ref[:, 1:, 0]

    pad_ref[...] = jnp.zeros_like(pad_ref)
    pad_ref[:, P:P + H, P:P + W] = s.reshape(NM, H, W).astype(jnp.bfloat16)
    for l in range(NM):
        acc = jnp.zeros((H, W), jnp.float32)
        for dy in range(_KSIZE):
            acc += jax.lax.dot_general(
                pad_ref[l, dy:dy + H, :], bands_ref[dy],
                (((1,), (0,)), ((), ())),
                preferred_element_type=jnp.float32)
        cm_ref[l] = acc

    bits = jax.lax.bitcast_convert_type(cm_ref[...], jnp.int32)

    def step(_, lohi):
        lo, hi = lohi
        mid = lo + (hi - lo + 1) // 2
        cnt = jnp.sum((bits >= mid).astype(jnp.int32), axis=(1, 2),
                      keepdims=True)
        big = cnt >= K
        return jnp.where(big, mid, lo), jnp.where(big, hi, mid - 1)

    lo0 = jnp.zeros((NM, 1, 1), jnp.int32)
    hi0 = jnp.full((NM, 1, 1), 0x3F800000, jnp.int32)
    lo, _ = jax.lax.fori_loop(0, 30, step, (lo0, hi0))

    sel = (bits >= lo).astype(jnp.float32)
    cnt = jnp.sum(sel, axis=(1, 2)).reshape(G, L - 1, 1)
    cnt_ref[:, 0] = jnp.zeros((G, cnt_ref.shape[2]), jnp.float32)
    cnt_ref[:, 1:] = jnp.broadcast_to(cnt, (G, L - 1, cnt_ref.shape[2]))
    mask_ref[:, 0] = jnp.ones((G, H, W), jnp.float32)
    mask_ref[:, 1:] = sel.reshape(G, L - 1, H, W)


def kernel(batch_confidence_maps, B, batch_warp_maks_list, record_len,
           warp_vis_list, warp_conf_list, warp_x_list, gauss_kernel):
    Bs, L, A, H, W = batch_confidence_maps.shape
    P = _KSIZE // 2

    gw = _gauss_2d_bf16()
    bands_np = np.zeros((_KSIZE, W + 2 * P, W), np.float32)
    cols = np.arange(W)
    for dy in range(_KSIZE):
        for dx in range(_KSIZE):
            bands_np[dy, cols + dx, cols] = gw[dy, dx]
    bands = jnp.asarray(bands_np, dtype=jnp.bfloat16)

    G = Bs if Bs <= 4 else (2 if Bs % 2 == 0 else 1)
    masks, counts = pl.pallas_call(
        _batch_body,
        grid=(Bs // G,),
        in_specs=[
            pl.BlockSpec((_KSIZE, W + 2 * P, W), lambda b: (0, 0, 0)),
            pl.BlockSpec((G, L, A, H, W), lambda b: (b, 0, 0, 0, 0)),
            pl.BlockSpec((G, L, 1, H, W), lambda b: (b, 0, 0, 0, 0)),
        ],
        out_specs=[
            pl.BlockSpec((G, L, H, W), lambda b: (b, 0, 0, 0)),
            pl.BlockSpec((G, L, 128), lambda b: (b, 0, 0)),
        ],
        out_shape=[
            jax.ShapeDtypeStruct((Bs, L, H, W), jnp.float32),
            jax.ShapeDtypeStruct((Bs, L, 128), jnp.float32),
        ],
        scratch_shapes=[
            pltpu.VMEM((G * (L - 1), H + 2 * P, W + 2 * P), jnp.bfloat16),
            pltpu.VMEM((G * (L - 1), H, W), jnp.float32),
        ],
        compiler_params=pltpu.CompilerParams(
            dimension_semantics=("arbitrary",)),
    )(bands, batch_confidence_maps, batch_warp_maks_list)

    masks = masks.reshape(Bs * L, 1, H, W)

    counts = counts[:, :, 0]
    rates = jnp.sum(counts[:, 1:], axis=1) / ((L - 1) * H * W)
    rate = jnp.sum(rates) / Bs
    return masks, rate
